# Initial kernel scaffold; baseline (speedup 1.0000x reference)
import jax
import jax.numpy as jnp
from jax.experimental import pallas as pl


def kernel(x, emb, W, b):
    raise NotImplementedError("write your pallas kernel here")


# speedup vs baseline: 6.8217x; 6.8217x over previous
import functools

import jax
import jax.numpy as jnp
from jax import lax
from jax.experimental import pallas as pl
from jax.experimental.pallas import tpu as pltpu
from jax.experimental.pallas import tpu_sc as plsc

_NC, _NS, _L = 2, 16, 16
_NW = _NC * _NS


def _fold_body(emb_ref, wr_ref, b_ref, t_ref):
    emb = emb_ref[:]
    rows = lax.broadcasted_iota(jnp.int32, (100, 7), 0)
    table = jnp.where(rows == 0, 0.0, emb)
    bias = b_ref[:] * 0.2
    for k in range(5):
        wk = wr_ref[k]
        tk = lax.dot_general(table, wk, (((1,), (1,)), ((), ())),
                             preferred_element_type=jnp.float32)
        t_ref[k] = tk + bias


def _make_sc_lookup(B):
    bpw = B // _NW
    groups = bpw // _L
    mesh = plsc.VectorSubcoreMesh(core_axis_name="c", subcore_axis_name="s")

    @functools.partial(
        pl.kernel,
        mesh=mesh,
        out_type=jax.ShapeDtypeStruct((B * 5,), jnp.float32),
        compiler_params=pltpu.CompilerParams(needs_layout_passes=False),
        scratch_types=[
            pltpu.VMEM((bpw * 5,), jnp.int32),
            pltpu.VMEM((2500,), jnp.float32),
            pltpu.VMEM((bpw * 5,), jnp.float32),
        ],
    )
    def body(x_hbm, t_hbm, out_hbm, xv, tv, ov):
        wid = lax.axis_index("s") * _NC + lax.axis_index("c")
        base5 = wid * (bpw * 5)
        pltpu.sync_copy(x_hbm.at[pl.ds(base5, bpw * 5)], xv)
        pltpu.sync_copy(t_hbm, tv)
        lane5 = lax.iota(jnp.int32, 16) * 5
        for g in range(groups):
            off = g * (_L * 5)
            acc = [None] * 5
            for k in range(5):
                xk = plsc.load_gather(xv, [lane5 + (off + k)])
                tb = xk * 5 + (k * 500)
                for j in range(5):
                    e = plsc.load_gather(tv, [tb + j])
                    acc[j] = e if k == 0 else acc[j] + e
            for j in range(5):
                o = 9.0 / (1.0 + jnp.exp(-acc[j]))
                plsc.store_scatter(ov, [lane5 + (off + j)], o)
        pltpu.sync_copy(ov, out_hbm.at[pl.ds(base5, bpw * 5)])

    return body


def kernel(x, emb, W, b):
    B = x.shape[0]
    assert B % (_NW * _L) == 0
    wr = W.reshape(5, 5, 7).transpose(1, 0, 2)
    t = pl.pallas_call(
        _fold_body,
        out_shape=jax.ShapeDtypeStruct((5, 100, 5), jnp.float32),
    )(emb, wr, b.reshape(1, 5))
    out_flat = _make_sc_lookup(B)(x.reshape(-1), t.reshape(-1))
    return out_flat.reshape(B, 5)


# speedup vs baseline: 7.4052x; 1.0855x over previous
import functools

import jax
import jax.numpy as jnp
from jax import lax
from jax.experimental import pallas as pl
from jax.experimental.pallas import tpu as pltpu
from jax.experimental.pallas import tpu_sc as plsc

_NC, _NS, _L = 2, 16, 16
_NW = _NC * _NS


def _fold_body(emb_ref, w_ref, b_ref, t_ref):
    emb = emb_ref[:]
    rows = lax.broadcasted_iota(jnp.int32, (100, 7), 0)
    table = jnp.where(rows == 0, 0.0, emb)
    bias = b_ref[:] * 0.2
    for k in range(5):
        wk = w_ref[:, 7 * k:7 * k + 7]
        tk = lax.dot_general(table, wk, (((1,), (1,)), ((), ())),
                             preferred_element_type=jnp.float32)
        t_ref[k] = tk + bias


def _make_sc_lookup(B):
    bpw = B // _NW
    groups = bpw // _L
    mesh = plsc.VectorSubcoreMesh(core_axis_name="c", subcore_axis_name="s")

    @functools.partial(
        pl.kernel,
        mesh=mesh,
        out_type=jax.ShapeDtypeStruct((B * 5,), jnp.float32),
        compiler_params=pltpu.CompilerParams(needs_layout_passes=False),
        scratch_types=[
            pltpu.VMEM((bpw * 5,), jnp.int32),
            pltpu.VMEM((2500,), jnp.float32),
            pltpu.VMEM((bpw * 5,), jnp.float32),
        ],
    )
    def body(x_hbm, t_hbm, out_hbm, xv, tv, ov):
        wid = lax.axis_index("s") * _NC + lax.axis_index("c")
        base5 = wid * (bpw * 5)
        pltpu.sync_copy(x_hbm.at[pl.ds(base5, bpw * 5)], xv)
        pltpu.sync_copy(t_hbm, tv)
        lane5 = lax.iota(jnp.int32, 16) * 5

        @plsc.parallel_loop(0, groups, unroll=4)
        def _group(g):
            idx0 = lane5 + g * (_L * 5)
            acc = [None] * 5
            for k in range(5):
                xk = plsc.load_gather(xv, [idx0 + k])
                tb = xk * 5 + (k * 500)
                for j in range(5):
                    e = plsc.load_gather(tv, [tb + j])
                    acc[j] = e if k == 0 else acc[j] + e
            for j in range(5):
                o = 9.0 / (1.0 + jnp.exp(-acc[j]))
                plsc.store_scatter(ov, [idx0 + j], o)
        pltpu.sync_copy(ov, out_hbm.at[pl.ds(base5, bpw * 5)])

    return body


def kernel(x, emb, W, b):
    B = x.shape[0]
    assert B % (_NW * _L) == 0
    t = pl.pallas_call(
        _fold_body,
        out_shape=jax.ShapeDtypeStruct((5, 100, 5), jnp.float32),
    )(emb, W, b.reshape(1, 5))
    out_flat = _make_sc_lookup(B)(x.reshape(-1), t.reshape(-1))
    return out_flat.reshape(B, 5)
